# Initial kernel scaffold; baseline (speedup 1.0000x reference)
import jax
import jax.numpy as jnp
from jax.experimental import pallas as pl


def kernel(x, edge_index, edge_attr, edge_weight, node_weight, W0, b0, W1, b1):
    raise NotImplementedError("write your pallas kernel here")


# speedup vs baseline: 2.1948x; 2.1948x over previous
import functools

import jax
import jax.numpy as jnp
from jax import lax
from jax.experimental import pallas as pl
from jax.experimental.pallas import tpu as pltpu
from jax.experimental.pallas import tpu_sc as plsc

N = 10000
E = 320000
D_EDGE = 16
D_NODE = 128
H = 128
HH = H // 2

NC = 2
NS = 16
NW = NC * NS
K = 80
NCHA = E // (NW * K)
NCHB = E // (NS * K)
NROW = 10240
RPT = NROW // NS
ZR = 128

_mesh = plsc.VectorSubcoreMesh(core_axis_name="c", subcore_axis_name="s",
                               num_cores=NC, num_subcores=NS)


def _zero_fill(zbuf, ncol16):
    z = jnp.zeros((16,), jnp.float32)

    def zrow(i, _):
        for c in range(ncol16):
            zbuf[i, pl.ds(c * 16, 16)] = z
        return 0

    lax.fori_loop(0, ZR, zrow, 0)


@functools.partial(
    pl.kernel,
    out_type=jax.ShapeDtypeStruct((NC, NROW, 2 * D_EDGE), jnp.float32),
    mesh=_mesh,
    compiler_params=pltpu.CompilerParams(use_tc_tiling_on_sc=False),
    scratch_types=[
        pltpu.VMEM((NCHA, K), jnp.int32),
        pltpu.VMEM((K, 16), jnp.float32),
        pltpu.VMEM((K, D_EDGE), jnp.float32),
        pltpu.VMEM((K, 2 * D_EDGE), jnp.float32),
        pltpu.VMEM((ZR, 2 * D_EDGE), jnp.float32),
        pltpu.VMEM_SHARED((NROW, 2 * D_EDGE), jnp.float32),
    ],
)
def _sc_edge_scatter(dst_hbm, attr_hbm, w_hbm, out_hbm,
                     dstv, wrepv, attrv, msgv, zbuf, acc):
    cid = lax.axis_index("c")
    sid = lax.axis_index("s")
    wid = cid * NS + sid

    _zero_fill(zbuf, 2)
    for b in range(RPT // ZR):
        pltpu.sync_copy(zbuf, acc.at[pl.ds(sid * RPT + b * ZR, ZR)])
    plsc.subcore_barrier()

    pltpu.sync_copy(dst_hbm.at[wid], dstv)

    lane = lax.iota(jnp.int32, 16)
    cvec = jnp.where(lane == 0, 1.0, 0.0).astype(jnp.float32)

    def chunk(j, _):
        pltpu.sync_copy(attr_hbm.at[wid, j], attrv)
        pltpu.sync_copy(w_hbm.at[wid, j], wrepv)

        def row(r, _):
            wbc = wrepv[r, pl.ds(0, 16)]
            msgv[r, pl.ds(0, 16)] = attrv[r, pl.ds(0, 16)] * wbc
            msgv[r, pl.ds(16, 16)] = cvec
            return 0

        lax.fori_loop(0, K, row, 0)
        pltpu.sync_copy(msgv, acc.at[dstv.at[j]], add=True)
        return 0

    lax.fori_loop(0, NCHA, chunk, 0)
    plsc.subcore_barrier()
    for b in range(RPT // ZR):
        sl = pl.ds(sid * RPT + b * ZR, ZR)
        pltpu.sync_copy(acc.at[sl], out_hbm.at[cid, sl])


@functools.partial(
    pl.kernel,
    out_type=jax.ShapeDtypeStruct((NC, NROW, HH), jnp.float32),
    mesh=_mesh,
    compiler_params=pltpu.CompilerParams(use_tc_tiling_on_sc=False),
    scratch_types=[
        pltpu.VMEM((NCHB, K), jnp.int32),
        pltpu.VMEM((NCHB, K), jnp.int32),
        pltpu.VMEM((K, 16), jnp.float32),
        pltpu.VMEM((K, HH), jnp.float32),
        pltpu.VMEM((ZR, HH), jnp.float32),
        pltpu.VMEM_SHARED((NROW, HH), jnp.float32),
        pltpu.SemaphoreType.DMA,
    ],
)
def _sc_msg_scatter(src_hbm, dst_hbm, w_hbm, h0pair_hbm, out_hbm,
                    srcv, dstv, wrepv, rows, zbuf, acc, sem):
    cid = lax.axis_index("c")
    sid = lax.axis_index("s")

    _zero_fill(zbuf, HH // 16)
    for b in range(RPT // ZR):
        pltpu.sync_copy(zbuf, acc.at[pl.ds(sid * RPT + b * ZR, ZR)])
    plsc.subcore_barrier()

    pltpu.sync_copy(src_hbm.at[sid], srcv)
    pltpu.sync_copy(dst_hbm.at[sid], dstv)

    def chunk(j, _):
        pltpu.async_copy(h0pair_hbm.at[cid].at[srcv.at[j]], rows, sem).wait()
        pltpu.sync_copy(w_hbm.at[sid, j], wrepv)

        def row(r, _):
            wbc = wrepv[r, pl.ds(0, 16)]
            for c in range(HH // 16):
                rows[r, pl.ds(c * 16, 16)] = rows[r, pl.ds(c * 16, 16)] * wbc
            return 0

        lax.fori_loop(0, K, row, 0)
        pltpu.sync_copy(rows, acc.at[dstv.at[j]], add=True)
        return 0

    lax.fori_loop(0, NCHB, chunk, 0)
    plsc.subcore_barrier()
    for b in range(RPT // ZR):
        sl = pl.ds(sid * RPT + b * ZR, ZR)
        pltpu.sync_copy(acc.at[sl], out_hbm.at[cid, sl])


_BLK = 1000


def _tc_h0_body(x_ref, pa_ref, w0_ref, b0_ref, o_ref, o2_ref):
    inc = pa_ref[0, :, 0:D_EDGE] + pa_ref[1, :, 0:D_EDGE]
    acc = lax.dot_general(x_ref[...], w0_ref[:, 0:D_NODE],
                          (((1,), (1,)), ((), ())),
                          preferred_element_type=jnp.float32)
    acc = acc + lax.dot_general(inc, w0_ref[:, D_NODE:D_NODE + D_EDGE],
                                (((1,), (1,)), ((), ())),
                                preferred_element_type=jnp.float32)
    h0 = jnp.maximum(acc + b0_ref[...], 0.0)
    o_ref[...] = h0
    o2_ref[0] = h0[:, 0:HH]
    o2_ref[1] = h0[:, HH:H]


def _tc_out_body(h0_ref, pb_ref, pa_ref, w1_ref, b1_ref, nw_ref, o_ref):
    cnt = pa_ref[0, :, D_EDGE:D_EDGE + 1] + pa_ref[1, :, D_EDGE:D_EDGE + 1]
    s = jnp.concatenate([pb_ref[0], pb_ref[1]], axis=1)
    aggr = s / jnp.maximum(cnt, 1.0)
    acc = lax.dot_general(aggr, w1_ref[...], (((1,), (1,)), ((), ())),
                          preferred_element_type=jnp.float32)
    o_ref[...] = jnp.maximum(h0_ref[...] + acc + b1_ref[...], 0.0) * nw_ref[...]


def kernel(x, edge_index, edge_attr, edge_weight, node_weight, W0, b0, W1, b1):
    dst_a = edge_index[1].reshape(NW, NCHA, K)
    attr_a = edge_attr.reshape(NW, NCHA, K, D_EDGE)
    wrep = jnp.broadcast_to(edge_weight[:, None], (E, 16))
    wrep_a = wrep.reshape(NW, NCHA, K, 16)

    part_a = _sc_edge_scatter(dst_a, attr_a, wrep_a)

    grid = N // _BLK
    h0, h0pair = pl.pallas_call(
        _tc_h0_body,
        grid=(grid,),
        in_specs=[
            pl.BlockSpec((_BLK, D_NODE), lambda i: (i, 0)),
            pl.BlockSpec((NC, _BLK, 2 * D_EDGE), lambda i: (0, i, 0)),
            pl.BlockSpec((H, D_NODE + D_EDGE), lambda i: (0, 0)),
            pl.BlockSpec((1, H), lambda i: (0, 0)),
        ],
        out_specs=[
            pl.BlockSpec((_BLK, H), lambda i: (i, 0)),
            pl.BlockSpec((NC, _BLK, HH), lambda i: (0, i, 0)),
        ],
        out_shape=[
            jax.ShapeDtypeStruct((N, H), jnp.float32),
            jax.ShapeDtypeStruct((NC, N, HH), jnp.float32),
        ],
    )(x, part_a, W0, b0.reshape(1, H))

    src_b = edge_index[0].reshape(NS, NCHB, K)
    dst_b = edge_index[1].reshape(NS, NCHB, K)
    wrep_b = wrep.reshape(NS, NCHB, K, 16)

    part_b = _sc_msg_scatter(src_b, dst_b, wrep_b, h0pair)

    h = pl.pallas_call(
        _tc_out_body,
        grid=(grid,),
        in_specs=[
            pl.BlockSpec((_BLK, H), lambda i: (i, 0)),
            pl.BlockSpec((NC, _BLK, HH), lambda i: (0, i, 0)),
            pl.BlockSpec((NC, _BLK, 2 * D_EDGE), lambda i: (0, i, 0)),
            pl.BlockSpec((H, H), lambda i: (0, 0)),
            pl.BlockSpec((1, H), lambda i: (0, 0)),
            pl.BlockSpec((_BLK, 1), lambda i: (i, 0)),
        ],
        out_specs=pl.BlockSpec((_BLK, H), lambda i: (i, 0)),
        out_shape=jax.ShapeDtypeStruct((N, H), jnp.float32),
    )(h0, part_b, part_a, W1, b1.reshape(1, H), node_weight.reshape(N, 1))

    return (h, h0)
